# Initial kernel scaffold; baseline (speedup 1.0000x reference)
import jax
import jax.numpy as jnp
from jax.experimental import pallas as pl


def kernel(x, edge_index, edge_weight, add_self_loops, W1, b1, W2, b2):
    raise NotImplementedError("write your pallas kernel here")


# speedup vs baseline: 4.7214x; 4.7214x over previous
import functools

import jax
import jax.numpy as jnp
from jax import lax
from jax.experimental import pallas as pl
from jax.experimental.pallas import tpu as pltpu
from jax.experimental.pallas import tpu_sc as plsc

N = 10000
D = 128
E = 320000
EP = 2 * 16 * 80 * 128
NP = 10240

_MESH = plsc.VectorSubcoreMesh(core_axis_name="c", subcore_axis_name="s")
_SC_PARAMS = pltpu.CompilerParams(needs_layout_passes=False)


def _mm1_body(x_ref, w_ref, o_ref):
    o_ref[...] = jnp.dot(x_ref[...], w_ref[...],
                         preferred_element_type=jnp.float32)


def _mm1(x, W):
    bn = 2000
    return pl.pallas_call(
        _mm1_body,
        grid=(N // bn,),
        in_specs=[
            pl.BlockSpec((bn, D), lambda i: (i, 0)),
            pl.BlockSpec((D, D), lambda i: (0, 0)),
        ],
        out_specs=pl.BlockSpec((bn, D), lambda i: (i, 0)),
        out_shape=jax.ShapeDtypeStruct((N, D), jnp.float32),
    )(x, W)


def _mm2_body(p0_ref, p1_ref, b_ref, w_ref, o_ref):
    h = jnp.maximum(p0_ref[...] + p1_ref[...] + b_ref[...], 0.0)
    o_ref[...] = jnp.dot(h, w_ref[...], preferred_element_type=jnp.float32)


def _mm2(p, b, W):
    bn = 2000
    nb = N // bn
    return pl.pallas_call(
        _mm2_body,
        grid=(nb,),
        in_specs=[
            pl.BlockSpec((bn, D), lambda i: (i, 0)),
            pl.BlockSpec((bn, D), lambda i: (nb + i, 0)),
            pl.BlockSpec((1, D), lambda i: (0, 0)),
            pl.BlockSpec((D, D), lambda i: (0, 0)),
        ],
        out_specs=pl.BlockSpec((bn, D), lambda i: (i, 0)),
        out_shape=jax.ShapeDtypeStruct((N, D), jnp.float32),
    )(p, p, b.reshape(1, D), W)


def _fin_body(p0_ref, p1_ref, b_ref, o_ref):
    o_ref[...] = p0_ref[...] + p1_ref[...] + b_ref[...]


def _fin(p, b):
    bn = 2000
    nb = N // bn
    return pl.pallas_call(
        _fin_body,
        grid=(nb,),
        in_specs=[
            pl.BlockSpec((bn, D), lambda i: (i, 0)),
            pl.BlockSpec((bn, D), lambda i: (nb + i, 0)),
            pl.BlockSpec((1, D), lambda i: (0, 0)),
        ],
        out_specs=pl.BlockSpec((bn, D), lambda i: (i, 0)),
        out_shape=jax.ShapeDtypeStruct((N, D), jnp.float32),
    )(p, p, b.reshape(1, D))


def _norm_body(row_hbm, ew_hbm, norm_hbm,
               degp, rbuf, wbuf, pbuf, dbuf, nbuf, deg_sh, dinv_sh, sem):
    c = lax.axis_index("c")
    s = lax.axis_index("s")
    NT = NP // 16

    def zero(i, _):
        degp[pl.ds(i * 16, 16)] = jnp.zeros((16,), jnp.float32)
        return _
    lax.fori_loop(0, NP // 16, zero, None)

    def dchunk(k, _):
        e0 = s * (EP // 16) + k * 2048
        pltpu.sync_copy(row_hbm.at[pl.ds(e0, 2048)], rbuf)
        pltpu.sync_copy(ew_hbm.at[pl.ds(e0, 2048)], wbuf)

        def body(i, _):
            r16 = rbuf[pl.ds(i * 16, 16)]
            w16 = wbuf[pl.ds(i * 16, 16)]
            plsc.addupdate_scatter(degp, [r16], w16)
            return _
        lax.fori_loop(0, 128, body, None)
        return _
    lax.fori_loop(0, (EP // 16) // 2048, dchunk, None)

    pltpu.sync_copy(degp, deg_sh.at[pl.ds(s * NP, NP)])
    plsc.subcore_barrier()

    n0 = s * NT
    for p in range(16):
        pltpu.sync_copy(deg_sh.at[pl.ds(p * NP + n0, NT)],
                        pbuf.at[pl.ds(p * NT, NT)])

    def red(k, _):
        acc = pbuf[pl.ds(k * 16, 16)]
        for p in range(1, 16):
            acc = acc + pbuf[pl.ds(p * NT + k * 16, 16)]
        dinv = jnp.where(acc == 0.0, 0.0, 1.0 / acc)
        dbuf[pl.ds(k * 16, 16)] = dinv
        return _
    lax.fori_loop(0, NT // 16, red, None)

    pltpu.sync_copy(dbuf, dinv_sh.at[pl.ds(n0, NT)])
    plsc.subcore_barrier()

    pltpu.sync_copy(dinv_sh, degp)
    wid = c * 16 + s

    def nchunk(k, _):
        e0 = wid * (EP // 32) + k * 2048
        pltpu.sync_copy(row_hbm.at[pl.ds(e0, 2048)], rbuf)
        pltpu.sync_copy(ew_hbm.at[pl.ds(e0, 2048)], wbuf)

        def body(i, _):
            r16 = rbuf[pl.ds(i * 16, 16)]
            w16 = wbuf[pl.ds(i * 16, 16)]
            d16 = plsc.load_gather(degp, [r16])
            nbuf[pl.ds(i * 16, 16)] = w16 * d16
            return _
        lax.fori_loop(0, 128, body, None)
        pltpu.sync_copy(nbuf, norm_hbm.at[pl.ds(e0, 2048)])
        return _
    lax.fori_loop(0, (EP // 32) // 2048, nchunk, None)


_norm_kernel = functools.partial(
    pl.kernel,
    out_type=jax.ShapeDtypeStruct((EP,), jnp.float32),
    mesh=_MESH,
    compiler_params=_SC_PARAMS,
    scratch_types=[
        pltpu.VMEM((NP,), jnp.float32),
        pltpu.VMEM((2048,), jnp.int32),
        pltpu.VMEM((2048,), jnp.float32),
        pltpu.VMEM((NP,), jnp.float32),
        pltpu.VMEM((NP // 16,), jnp.float32),
        pltpu.VMEM((2048,), jnp.float32),
        pltpu.VMEM_SHARED((16 * NP,), jnp.float32),
        pltpu.VMEM_SHARED((NP,), jnp.float32),
        pltpu.SemaphoreType.DMA,
    ],
)(_norm_body)


TPT = 80
CH = 2
NCH = TPT // CH
RPT = 640
RLAST = N - 15 * RPT


def _agg_body(h_hbm, row_hbm, col_hbm, norm_hbm, out_hbm,
              rlin, clin, nbuf, ridx, cidx, gbuf, acc, sem):
    c = lax.axis_index("c")
    s = lax.axis_index("s")
    cN = c * N
    r0 = s * RPT

    def zrow(r, _):
        for q in range(8):
            gbuf[r, pl.ds(q * 16, 16)] = jnp.zeros((16,), jnp.float32)
        return _
    lax.fori_loop(0, CH * 128, zrow, None)

    @pl.when(s < 15)
    def _():
        for t in range((RPT + CH * 128 - 1) // (CH * 128)):
            nr = min(CH * 128, RPT - t * CH * 128)
            pltpu.sync_copy(gbuf.at[pl.ds(0, nr)],
                            acc.at[pl.ds(r0 + t * CH * 128, nr)])

    @pl.when(s == 15)
    def _():
        for t in range((RLAST + CH * 128 - 1) // (CH * 128)):
            nr = min(CH * 128, RLAST - t * CH * 128)
            pltpu.sync_copy(gbuf.at[pl.ds(0, nr)],
                            acc.at[pl.ds(15 * RPT + t * CH * 128, nr)])
    plsc.subcore_barrier()

    def chunk(k, _):
        e0 = c * (EP // 2) + s * (TPT * 128) + k * (CH * 128)
        pltpu.sync_copy(row_hbm.at[pl.ds(e0, CH * 128)], rlin)
        pltpu.sync_copy(col_hbm.at[pl.ds(e0, CH * 128)], clin)
        pltpu.sync_copy(norm_hbm.at[pl.ds(e0, CH * 128)], nbuf)
        for i in range(CH * 8):
            ridx[i // 8, pl.ds((i % 8) * 16, 16)] = rlin[pl.ds(i * 16, 16)]
            cidx[i // 8, pl.ds((i % 8) * 16, 16)] = clin[pl.ds(i * 16, 16)]
        cps = [pltpu.async_copy(h_hbm.at[ridx.at[j]],
                                gbuf.at[pl.ds(j * 128, 128)], sem)
               for j in range(CH)]
        for cp in cps:
            cp.wait()

        def sgrp(i, _):
            n16 = nbuf[pl.ds(i * 16, 16)]
            for u in range(16):
                j = i * 16 + u
                nb = n16[u]
                for q in range(8):
                    gbuf[j, pl.ds(q * 16, 16)] = (
                        gbuf[j, pl.ds(q * 16, 16)] * nb)
            return _
        lax.fori_loop(0, CH * 8, sgrp, None)

        for j in range(CH):
            pltpu.sync_copy(gbuf.at[pl.ds(j * 128, 128)],
                            acc.at[cidx.at[j]], add=True)
        return _
    lax.fori_loop(0, NCH, chunk, None)
    plsc.subcore_barrier()

    @pl.when(s < 15)
    def _():
        pltpu.sync_copy(acc.at[pl.ds(r0, RPT)],
                        out_hbm.at[pl.ds(cN + r0, RPT)])

    @pl.when(s == 15)
    def _():
        pltpu.sync_copy(acc.at[pl.ds(15 * RPT, RLAST)],
                        out_hbm.at[pl.ds(cN + 15 * RPT, RLAST)])


_agg_kernel = functools.partial(
    pl.kernel,
    out_type=jax.ShapeDtypeStruct((2 * N, D), jnp.float32),
    mesh=_MESH,
    compiler_params=_SC_PARAMS,
    scratch_types=[
        pltpu.VMEM((CH * 128,), jnp.int32),
        pltpu.VMEM((CH * 128,), jnp.int32),
        pltpu.VMEM((CH * 128,), jnp.float32),
        pltpu.VMEM((CH, 128), jnp.int32),
        pltpu.VMEM((CH, 128), jnp.int32),
        pltpu.VMEM((CH * 128, D), jnp.float32),
        pltpu.VMEM_SHARED((N, D), jnp.float32),
        pltpu.SemaphoreType.DMA,
    ],
)(_agg_body)


def kernel(x, edge_index, edge_weight, add_self_loops, W1, b1, W2, b2):
    del add_self_loops
    pad = EP - E
    row_p = jnp.pad(edge_index[0], (0, pad))
    col_p = jnp.pad(edge_index[1], (0, pad))
    ew_p = jnp.pad(edge_weight, (0, pad))

    norm = _norm_kernel(row_p, ew_p)
    h1 = _mm1(x, W1)
    p1 = _agg_kernel(h1, row_p, col_p, norm)
    h2 = _mm2(p1, b1, W2)
    p2 = _agg_kernel(h2, row_p, col_p, norm)
    return _fin(p2, b2)


# speedup vs baseline: 10.2300x; 2.1667x over previous
import functools

import jax
import jax.numpy as jnp
from jax import lax
from jax.experimental import pallas as pl
from jax.experimental.pallas import tpu as pltpu
from jax.experimental.pallas import tpu_sc as plsc

N = 10000
D = 128
E = 320000
EP = 2 * 16 * 80 * 128
NP = 10240

_MESH = plsc.VectorSubcoreMesh(core_axis_name="c", subcore_axis_name="s")
_SC_PARAMS = pltpu.CompilerParams(needs_layout_passes=False)


def _mm1_body(x_ref, w_ref, o_ref):
    o_ref[...] = jnp.dot(x_ref[...], w_ref[...],
                         preferred_element_type=jnp.float32)


def _mm1(x, W):
    bn = 2000
    return pl.pallas_call(
        _mm1_body,
        grid=(N // bn,),
        in_specs=[
            pl.BlockSpec((bn, D), lambda i: (i, 0)),
            pl.BlockSpec((D, D), lambda i: (0, 0)),
        ],
        out_specs=pl.BlockSpec((bn, D), lambda i: (i, 0)),
        out_shape=jax.ShapeDtypeStruct((N, D), jnp.float32),
    )(x, W)


def _mm2_body(p0_ref, p1_ref, b_ref, w_ref, o_ref):
    h = jnp.maximum(p0_ref[...] + p1_ref[...] + b_ref[...], 0.0)
    o_ref[...] = jnp.dot(h, w_ref[...], preferred_element_type=jnp.float32)


def _mm2(p, b, W):
    bn = 2000
    nb = N // bn
    return pl.pallas_call(
        _mm2_body,
        grid=(nb,),
        in_specs=[
            pl.BlockSpec((bn, D), lambda i: (i, 0)),
            pl.BlockSpec((bn, D), lambda i: (nb + i, 0)),
            pl.BlockSpec((1, D), lambda i: (0, 0)),
            pl.BlockSpec((D, D), lambda i: (0, 0)),
        ],
        out_specs=pl.BlockSpec((bn, D), lambda i: (i, 0)),
        out_shape=jax.ShapeDtypeStruct((N, D), jnp.float32),
    )(p, p, b.reshape(1, D), W)


def _fin_body(p0_ref, p1_ref, b_ref, o_ref):
    o_ref[...] = p0_ref[...] + p1_ref[...] + b_ref[...]


def _fin(p, b):
    bn = 2000
    nb = N // bn
    return pl.pallas_call(
        _fin_body,
        grid=(nb,),
        in_specs=[
            pl.BlockSpec((bn, D), lambda i: (i, 0)),
            pl.BlockSpec((bn, D), lambda i: (nb + i, 0)),
            pl.BlockSpec((1, D), lambda i: (0, 0)),
        ],
        out_specs=pl.BlockSpec((bn, D), lambda i: (i, 0)),
        out_shape=jax.ShapeDtypeStruct((N, D), jnp.float32),
    )(p, p, b.reshape(1, D))


def _norm_body(row_hbm, ew_hbm, norm_hbm,
               degp, rbuf, wbuf, pbuf, dbuf, nbuf, deg_sh, dinv_sh, sem):
    c = lax.axis_index("c")
    s = lax.axis_index("s")
    NT = NP // 16

    def zero(i, _):
        degp[pl.ds(i * 16, 16)] = jnp.zeros((16,), jnp.float32)
        return _
    lax.fori_loop(0, NP // 16, zero, None)

    def dchunk(k, _):
        e0 = s * (EP // 16) + k * 2048
        pltpu.sync_copy(row_hbm.at[pl.ds(e0, 2048)], rbuf)
        pltpu.sync_copy(ew_hbm.at[pl.ds(e0, 2048)], wbuf)

        def body(i, _):
            r16 = rbuf[pl.ds(i * 16, 16)]
            w16 = wbuf[pl.ds(i * 16, 16)]
            plsc.addupdate_scatter(degp, [r16], w16)
            return _
        lax.fori_loop(0, 128, body, None)
        return _
    lax.fori_loop(0, (EP // 16) // 2048, dchunk, None)

    pltpu.sync_copy(degp, deg_sh.at[pl.ds(s * NP, NP)])
    plsc.subcore_barrier()

    n0 = s * NT
    for p in range(16):
        pltpu.sync_copy(deg_sh.at[pl.ds(p * NP + n0, NT)],
                        pbuf.at[pl.ds(p * NT, NT)])

    def red(k, _):
        acc = pbuf[pl.ds(k * 16, 16)]
        for p in range(1, 16):
            acc = acc + pbuf[pl.ds(p * NT + k * 16, 16)]
        dinv = jnp.where(acc == 0.0, 0.0, 1.0 / acc)
        dbuf[pl.ds(k * 16, 16)] = dinv
        return _
    lax.fori_loop(0, NT // 16, red, None)

    pltpu.sync_copy(dbuf, dinv_sh.at[pl.ds(n0, NT)])
    plsc.subcore_barrier()

    pltpu.sync_copy(dinv_sh, degp)
    wid = c * 16 + s

    def nchunk(k, _):
        e0 = wid * (EP // 32) + k * 2048
        pltpu.sync_copy(row_hbm.at[pl.ds(e0, 2048)], rbuf)
        pltpu.sync_copy(ew_hbm.at[pl.ds(e0, 2048)], wbuf)

        def body(i, _):
            r16 = rbuf[pl.ds(i * 16, 16)]
            w16 = wbuf[pl.ds(i * 16, 16)]
            d16 = plsc.load_gather(degp, [r16])
            nbuf[pl.ds(i * 16, 16)] = w16 * d16
            return _
        lax.fori_loop(0, 128, body, None)
        pltpu.sync_copy(nbuf, norm_hbm.at[pl.ds(e0, 2048)])
        return _
    lax.fori_loop(0, (EP // 32) // 2048, nchunk, None)


_norm_kernel = functools.partial(
    pl.kernel,
    out_type=jax.ShapeDtypeStruct((EP,), jnp.float32),
    mesh=_MESH,
    compiler_params=_SC_PARAMS,
    scratch_types=[
        pltpu.VMEM((NP,), jnp.float32),
        pltpu.VMEM((2048,), jnp.int32),
        pltpu.VMEM((2048,), jnp.float32),
        pltpu.VMEM((NP,), jnp.float32),
        pltpu.VMEM((NP // 16,), jnp.float32),
        pltpu.VMEM((2048,), jnp.float32),
        pltpu.VMEM_SHARED((16 * NP,), jnp.float32),
        pltpu.VMEM_SHARED((NP,), jnp.float32),
        pltpu.SemaphoreType.DMA,
    ],
)(_norm_body)


TPT = 80
CH = 2
NCH = TPT // CH
RPT = 640
RLAST = N - 15 * RPT


def _agg_body(h_hbm, row_hbm, col_hbm, norm_hbm, out_hbm,
              rlin, clin, nbuf, ridx, cidx, gbuf, acc, sem):
    c = lax.axis_index("c")
    s = lax.axis_index("s")
    cN = c * N
    r0 = s * RPT

    def zrow(r, _):
        for q in range(8):
            gbuf[r, pl.ds(q * 16, 16)] = jnp.zeros((16,), jnp.float32)
        return _
    lax.fori_loop(0, CH * 128, zrow, None)

    @pl.when(s < 15)
    def _():
        for t in range((RPT + CH * 128 - 1) // (CH * 128)):
            nr = min(CH * 128, RPT - t * CH * 128)
            pltpu.sync_copy(gbuf.at[pl.ds(0, nr)],
                            acc.at[pl.ds(r0 + t * CH * 128, nr)])

    @pl.when(s == 15)
    def _():
        for t in range((RLAST + CH * 128 - 1) // (CH * 128)):
            nr = min(CH * 128, RLAST - t * CH * 128)
            pltpu.sync_copy(gbuf.at[pl.ds(0, nr)],
                            acc.at[pl.ds(15 * RPT + t * CH * 128, nr)])
    plsc.subcore_barrier()

    def chunk(k, _):
        e0 = c * (EP // 2) + s * (TPT * 128) + k * (CH * 128)
        pltpu.sync_copy(row_hbm.at[pl.ds(e0, CH * 128)], rlin)
        pltpu.sync_copy(col_hbm.at[pl.ds(e0, CH * 128)], clin)
        pltpu.sync_copy(norm_hbm.at[pl.ds(e0, CH * 128)], nbuf)
        for i in range(CH * 8):
            ridx[i // 8, pl.ds((i % 8) * 16, 16)] = rlin[pl.ds(i * 16, 16)]
            cidx[i // 8, pl.ds((i % 8) * 16, 16)] = clin[pl.ds(i * 16, 16)]
        cps = [pltpu.async_copy(h_hbm.at[ridx.at[j]],
                                gbuf.at[pl.ds(j * 128, 128)], sem)
               for j in range(CH)]
        for cp in cps:
            cp.wait()

        def sgrp(i, _):
            n16 = nbuf[pl.ds(i * 16, 16)]
            for u in range(16):
                j = i * 16 + u
                nb = n16[u]
                for q in range(8):
                    gbuf[j, pl.ds(q * 16, 16)] = (
                        gbuf[j, pl.ds(q * 16, 16)] * nb)
            return _
        lax.fori_loop(0, CH * 8, sgrp, None)

        for j in range(CH):
            pltpu.sync_copy(gbuf.at[pl.ds(j * 128, 128)],
                            acc.at[cidx.at[j]], add=True)
        return _
    lax.fori_loop(0, NCH, chunk, None)
    plsc.subcore_barrier()

    @pl.when(s < 15)
    def _():
        pltpu.sync_copy(acc.at[pl.ds(r0, RPT)],
                        out_hbm.at[pl.ds(cN + r0, RPT)])

    @pl.when(s == 15)
    def _():
        pltpu.sync_copy(acc.at[pl.ds(15 * RPT, RLAST)],
                        out_hbm.at[pl.ds(cN + 15 * RPT, RLAST)])


_agg_kernel = functools.partial(
    pl.kernel,
    out_type=jax.ShapeDtypeStruct((2 * N, D), jnp.float32),
    mesh=_MESH,
    compiler_params=_SC_PARAMS,
    scratch_types=[
        pltpu.VMEM((CH * 128,), jnp.int32),
        pltpu.VMEM((CH * 128,), jnp.int32),
        pltpu.VMEM((CH * 128,), jnp.float32),
        pltpu.VMEM((CH, 128), jnp.int32),
        pltpu.VMEM((CH, 128), jnp.int32),
        pltpu.VMEM((CH * 128, D), jnp.float32),
        pltpu.VMEM_SHARED((N, D), jnp.float32),
        pltpu.SemaphoreType.DMA,
    ],
)(_agg_body)


def kernel(x, edge_index, edge_weight, add_self_loops, W1, b1, W2, b2):
    del add_self_loops
    pad = EP - E
    spread = (jnp.arange(pad, dtype=jnp.int32) * 16) % N
    row_p = jnp.concatenate([edge_index[0], spread])
    col_p = jnp.concatenate([edge_index[1], spread])
    ew_p = jnp.pad(edge_weight, (0, pad))

    norm = _norm_kernel(row_p, ew_p)
    h1 = _mm1(x, W1)
    p1 = _agg_kernel(h1, row_p, col_p, norm)
    h2 = _mm2(p1, b1, W2)
    p2 = _agg_kernel(h2, row_p, col_p, norm)
    return _fin(p2, b2)


# speedup vs baseline: 15.9957x; 1.5636x over previous
import functools

import jax
import jax.numpy as jnp
from jax import lax
from jax.experimental import pallas as pl
from jax.experimental.pallas import tpu as pltpu
from jax.experimental.pallas import tpu_sc as plsc

N = 10000
D = 128
E = 320000
EP = 2 * 16 * 80 * 128
NP = 10240

_MESH = plsc.VectorSubcoreMesh(core_axis_name="c", subcore_axis_name="s")
_SC_PARAMS = pltpu.CompilerParams(needs_layout_passes=False)


def _mm1_body(x_ref, w_ref, o_ref):
    o_ref[...] = jnp.dot(x_ref[...], w_ref[...],
                         preferred_element_type=jnp.float32)


def _mm1(x, W):
    bn = 2000
    return pl.pallas_call(
        _mm1_body,
        grid=(N // bn,),
        in_specs=[
            pl.BlockSpec((bn, D), lambda i: (i, 0)),
            pl.BlockSpec((D, D), lambda i: (0, 0)),
        ],
        out_specs=pl.BlockSpec((bn, D), lambda i: (i, 0)),
        out_shape=jax.ShapeDtypeStruct((N, D), jnp.float32),
    )(x, W)


def _mm2_body(p0_ref, p1_ref, b_ref, w_ref, o_ref):
    h = jnp.maximum(p0_ref[...] + p1_ref[...] + b_ref[...], 0.0)
    o_ref[...] = jnp.dot(h, w_ref[...], preferred_element_type=jnp.float32)


def _mm2(p, b, W):
    bn = 2000
    nb = N // bn
    return pl.pallas_call(
        _mm2_body,
        grid=(nb,),
        in_specs=[
            pl.BlockSpec((bn, D), lambda i: (i, 0)),
            pl.BlockSpec((bn, D), lambda i: (nb + i, 0)),
            pl.BlockSpec((1, D), lambda i: (0, 0)),
            pl.BlockSpec((D, D), lambda i: (0, 0)),
        ],
        out_specs=pl.BlockSpec((bn, D), lambda i: (i, 0)),
        out_shape=jax.ShapeDtypeStruct((N, D), jnp.float32),
    )(p, p, b.reshape(1, D), W)


def _fin_body(p0_ref, p1_ref, b_ref, o_ref):
    o_ref[...] = p0_ref[...] + p1_ref[...] + b_ref[...]


def _fin(p, b):
    bn = 2000
    nb = N // bn
    return pl.pallas_call(
        _fin_body,
        grid=(nb,),
        in_specs=[
            pl.BlockSpec((bn, D), lambda i: (i, 0)),
            pl.BlockSpec((bn, D), lambda i: (nb + i, 0)),
            pl.BlockSpec((1, D), lambda i: (0, 0)),
        ],
        out_specs=pl.BlockSpec((bn, D), lambda i: (i, 0)),
        out_shape=jax.ShapeDtypeStruct((N, D), jnp.float32),
    )(p, p, b.reshape(1, D))


def _norm_body(row_hbm, ew_hbm, norm_hbm,
               degp, rbuf, wbuf, pbuf, dbuf, nbuf, deg_sh, dinv_sh, sem):
    c = lax.axis_index("c")
    s = lax.axis_index("s")
    NT = NP // 16

    def zero(i, _):
        degp[pl.ds(i * 16, 16)] = jnp.zeros((16,), jnp.float32)
        return _
    lax.fori_loop(0, NP // 16, zero, None)

    def dchunk(k, _):
        e0 = s * (EP // 16) + k * 2048
        pltpu.sync_copy(row_hbm.at[pl.ds(e0, 2048)], rbuf)
        pltpu.sync_copy(ew_hbm.at[pl.ds(e0, 2048)], wbuf)

        def body(i, _):
            r16 = rbuf[pl.ds(i * 16, 16)]
            w16 = wbuf[pl.ds(i * 16, 16)]
            plsc.addupdate_scatter(degp, [r16], w16)
            return _
        lax.fori_loop(0, 128, body, None)
        return _
    lax.fori_loop(0, (EP // 16) // 2048, dchunk, None)

    pltpu.sync_copy(degp, deg_sh.at[pl.ds(s * NP, NP)])
    plsc.subcore_barrier()

    n0 = s * NT
    for p in range(16):
        pltpu.sync_copy(deg_sh.at[pl.ds(p * NP + n0, NT)],
                        pbuf.at[pl.ds(p * NT, NT)])

    def red(k, _):
        acc = pbuf[pl.ds(k * 16, 16)]
        for p in range(1, 16):
            acc = acc + pbuf[pl.ds(p * NT + k * 16, 16)]
        dinv = jnp.where(acc == 0.0, 0.0, 1.0 / acc)
        dbuf[pl.ds(k * 16, 16)] = dinv
        return _
    lax.fori_loop(0, NT // 16, red, None)

    pltpu.sync_copy(dbuf, dinv_sh.at[pl.ds(n0, NT)])
    plsc.subcore_barrier()

    pltpu.sync_copy(dinv_sh, degp)
    wid = c * 16 + s

    def nchunk(k, _):
        e0 = wid * (EP // 32) + k * 2048
        pltpu.sync_copy(row_hbm.at[pl.ds(e0, 2048)], rbuf)
        pltpu.sync_copy(ew_hbm.at[pl.ds(e0, 2048)], wbuf)

        def body(i, _):
            r16 = rbuf[pl.ds(i * 16, 16)]
            w16 = wbuf[pl.ds(i * 16, 16)]
            d16 = plsc.load_gather(degp, [r16])
            nbuf[pl.ds(i * 16, 16)] = w16 * d16
            return _
        lax.fori_loop(0, 128, body, None)
        pltpu.sync_copy(nbuf, norm_hbm.at[pl.ds(e0, 2048)])
        return _
    lax.fori_loop(0, (EP // 32) // 2048, nchunk, None)


_norm_kernel = functools.partial(
    pl.kernel,
    out_type=jax.ShapeDtypeStruct((EP,), jnp.float32),
    mesh=_MESH,
    compiler_params=_SC_PARAMS,
    scratch_types=[
        pltpu.VMEM((NP,), jnp.float32),
        pltpu.VMEM((2048,), jnp.int32),
        pltpu.VMEM((2048,), jnp.float32),
        pltpu.VMEM((NP,), jnp.float32),
        pltpu.VMEM((NP // 16,), jnp.float32),
        pltpu.VMEM((2048,), jnp.float32),
        pltpu.VMEM_SHARED((16 * NP,), jnp.float32),
        pltpu.VMEM_SHARED((NP,), jnp.float32),
        pltpu.SemaphoreType.DMA,
    ],
)(_norm_body)


TPT = 80
GRP = 4
PAIR = 2 * GRP
NPAIR = TPT // PAIR
RPT = 640
RLAST = N - 15 * RPT


def _agg_body(h_hbm, row_hbm, col_hbm, norm_hbm, out_hbm,
              rlin0, clin0, nbuf0, ridx0, cidx0,
              rlin1, clin1, nbuf1, ridx1, cidx1,
              gbufA, gbufB, acc, sem_l, sem_a, sem_b):
    c = lax.axis_index("c")
    s = lax.axis_index("s")
    r0 = s * RPT
    L = [(rlin0, clin0, nbuf0, ridx0, cidx0),
         (rlin1, clin1, nbuf1, ridx1, cidx1)]
    G = [gbufA, gbufB]
    SG = [sem_a, sem_b]

    def zrow(r, _):
        for q in range(8):
            gbufA[r, pl.ds(q * 16, 16)] = jnp.zeros((16,), jnp.float32)
        return _
    lax.fori_loop(0, 128, zrow, None)

    @pl.when(s < 15)
    def _():
        for t in range(RPT // 128):
            pltpu.sync_copy(gbufA, acc.at[pl.ds(r0 + t * 128, 128)])

    @pl.when(s == 15)
    def _():
        for t in range((RLAST + 127) // 128):
            nr = min(128, RLAST - t * 128)
            pltpu.sync_copy(gbufA.at[pl.ds(0, nr)],
                            acc.at[pl.ds(15 * RPT + t * 128, nr)])
    plsc.subcore_barrier()

    ebase = c * (EP // 2) + s * (TPT * 128)

    def _load_group(g, gb):
        rlin, clin, nbuf, ridx, cidx = L[gb]
        e0 = ebase + g * (GRP * 128)
        cps = [pltpu.async_copy(row_hbm.at[pl.ds(e0, GRP * 128)], rlin, sem_l),
               pltpu.async_copy(col_hbm.at[pl.ds(e0, GRP * 128)], clin, sem_l),
               pltpu.async_copy(norm_hbm.at[pl.ds(e0, GRP * 128)], nbuf, sem_l)]
        for cp in cps:
            cp.wait()
        for i in range(GRP * 8):
            ridx[i // 8, pl.ds((i % 8) * 16, 16)] = rlin[pl.ds(i * 16, 16)]
            cidx[i // 8, pl.ds((i % 8) * 16, 16)] = clin[pl.ds(i * 16, 16)]

    def _gather(gb, t, b):
        ridx = L[gb][3]
        pltpu.async_copy(h_hbm.at[ridx.at[t]], G[b], SG[b])

    def _process(gb, t, b):
        nbuf, cidx = L[gb][2], L[gb][4]
        gbuf = G[b]
        pltpu.make_async_copy(h_hbm.at[L[gb][3].at[t]], gbuf, SG[b]).wait()

        def sgrp(i, _):
            n16 = nbuf[pl.ds(t * 128 + i * 16, 16)]
            for u in range(16):
                j = i * 16 + u
                nb = n16[u]
                for q in range(8):
                    gbuf[j, pl.ds(q * 16, 16)] = (
                        gbuf[j, pl.ds(q * 16, 16)] * nb)
            return _
        lax.fori_loop(0, 8, sgrp, None)
        pltpu.sync_copy(gbuf, acc.at[cidx.at[t]], add=True)

    def pair(i, _):
        for gb in range(2):
            g = 2 * i + gb
            _load_group(g, gb)
            for t in range(GRP):
                u = gb * GRP + t
                b = u % 2
                _gather(gb, t, b)
                pu = u - 1
                pgb, pt, pb = (pu // GRP) % 2, pu % GRP, pu % 2
                if u == 0:
                    @pl.when(i > 0)
                    def _():
                        _process(pgb, pt, pb)
                else:
                    _process(pgb, pt, pb)
        return _
    lax.fori_loop(0, NPAIR, pair, None)
    _process(1, GRP - 1, (PAIR - 1) % 2)
    plsc.subcore_barrier()

    @pl.when(s < 15)
    def _():
        pltpu.sync_copy(acc.at[pl.ds(r0, RPT)],
                        out_hbm.at[pl.ds(c * N + r0, RPT)])

    @pl.when(s == 15)
    def _():
        pltpu.sync_copy(acc.at[pl.ds(15 * RPT, RLAST)],
                        out_hbm.at[pl.ds(c * N + 15 * RPT, RLAST)])


_agg_kernel = functools.partial(
    pl.kernel,
    out_type=jax.ShapeDtypeStruct((2 * N, D), jnp.float32),
    mesh=_MESH,
    compiler_params=_SC_PARAMS,
    scratch_types=[
        pltpu.VMEM((GRP * 128,), jnp.int32),
        pltpu.VMEM((GRP * 128,), jnp.int32),
        pltpu.VMEM((GRP * 128,), jnp.float32),
        pltpu.VMEM((GRP, 128), jnp.int32),
        pltpu.VMEM((GRP, 128), jnp.int32),
        pltpu.VMEM((GRP * 128,), jnp.int32),
        pltpu.VMEM((GRP * 128,), jnp.int32),
        pltpu.VMEM((GRP * 128,), jnp.float32),
        pltpu.VMEM((GRP, 128), jnp.int32),
        pltpu.VMEM((GRP, 128), jnp.int32),
        pltpu.VMEM((128, D), jnp.float32),
        pltpu.VMEM((128, D), jnp.float32),
        pltpu.VMEM_SHARED((N, D), jnp.float32),
        pltpu.SemaphoreType.DMA,
        pltpu.SemaphoreType.DMA,
        pltpu.SemaphoreType.DMA,
    ],
)(_agg_body)


def kernel(x, edge_index, edge_weight, add_self_loops, W1, b1, W2, b2):
    del add_self_loops
    pad = EP - E
    spread = (jnp.arange(pad, dtype=jnp.int32) * 16) % N
    row_p = jnp.concatenate([edge_index[0], spread])
    col_p = jnp.concatenate([edge_index[1], spread])
    ew_p = jnp.pad(edge_weight, (0, pad))

    norm = _norm_kernel(row_p, ew_p)
    h1 = _mm1(x, W1)
    p1 = _agg_kernel(h1, row_p, col_p, norm)
    h2 = _mm2(p1, b1, W2)
    p2 = _agg_kernel(h2, row_p, col_p, norm)
    return _fin(p2, b2)


# speedup vs baseline: 16.8381x; 1.0527x over previous
import functools

import jax
import jax.numpy as jnp
from jax import lax
from jax.experimental import pallas as pl
from jax.experimental.pallas import tpu as pltpu
from jax.experimental.pallas import tpu_sc as plsc

N = 10000
D = 128
E = 320000
EP = 2 * 16 * 80 * 128
NP = 10240

_MESH = plsc.VectorSubcoreMesh(core_axis_name="c", subcore_axis_name="s")
_SC_PARAMS = pltpu.CompilerParams(needs_layout_passes=False)


def _mm1_body(x_ref, w_ref, o_ref):
    o_ref[...] = jnp.dot(x_ref[...], w_ref[...],
                         preferred_element_type=jnp.float32)


def _mm1(x, W):
    bn = 2000
    return pl.pallas_call(
        _mm1_body,
        grid=(N // bn,),
        in_specs=[
            pl.BlockSpec((bn, D), lambda i: (i, 0)),
            pl.BlockSpec((D, D), lambda i: (0, 0)),
        ],
        out_specs=pl.BlockSpec((bn, D), lambda i: (i, 0)),
        out_shape=jax.ShapeDtypeStruct((N, D), jnp.float32),
    )(x, W)


def _mm2_body(p0_ref, p1_ref, b_ref, w_ref, o_ref):
    h = jnp.maximum(p0_ref[...] + p1_ref[...] + b_ref[...], 0.0)
    o_ref[...] = jnp.dot(h, w_ref[...], preferred_element_type=jnp.float32)


def _mm2(p, b, W):
    bn = 2000
    nb = N // bn
    return pl.pallas_call(
        _mm2_body,
        grid=(nb,),
        in_specs=[
            pl.BlockSpec((bn, D), lambda i: (i, 0)),
            pl.BlockSpec((bn, D), lambda i: (nb + i, 0)),
            pl.BlockSpec((1, D), lambda i: (0, 0)),
            pl.BlockSpec((D, D), lambda i: (0, 0)),
        ],
        out_specs=pl.BlockSpec((bn, D), lambda i: (i, 0)),
        out_shape=jax.ShapeDtypeStruct((N, D), jnp.float32),
    )(p, p, b.reshape(1, D), W)


def _fin_body(p0_ref, p1_ref, b_ref, o_ref):
    o_ref[...] = p0_ref[...] + p1_ref[...] + b_ref[...]


def _fin(p, b):
    bn = 2000
    nb = N // bn
    return pl.pallas_call(
        _fin_body,
        grid=(nb,),
        in_specs=[
            pl.BlockSpec((bn, D), lambda i: (i, 0)),
            pl.BlockSpec((bn, D), lambda i: (nb + i, 0)),
            pl.BlockSpec((1, D), lambda i: (0, 0)),
        ],
        out_specs=pl.BlockSpec((bn, D), lambda i: (i, 0)),
        out_shape=jax.ShapeDtypeStruct((N, D), jnp.float32),
    )(p, p, b.reshape(1, D))


def _norm_body(row_hbm, ew_hbm, norm_hbm,
               degp, rbuf, wbuf, pbuf, dbuf, nbuf, deg_sh, dinv_sh, sem):
    c = lax.axis_index("c")
    s = lax.axis_index("s")
    NT = NP // 16

    def zero(i, _):
        degp[pl.ds(i * 16, 16)] = jnp.zeros((16,), jnp.float32)
        return _
    lax.fori_loop(0, NP // 16, zero, None)

    def dchunk(k, _):
        e0 = s * (EP // 16) + k * 2048
        pltpu.sync_copy(row_hbm.at[pl.ds(e0, 2048)], rbuf)
        pltpu.sync_copy(ew_hbm.at[pl.ds(e0, 2048)], wbuf)

        def body(i, _):
            r16 = rbuf[pl.ds(i * 16, 16)]
            w16 = wbuf[pl.ds(i * 16, 16)]
            plsc.addupdate_scatter(degp, [r16], w16)
            return _
        lax.fori_loop(0, 128, body, None)
        return _
    lax.fori_loop(0, (EP // 16) // 2048, dchunk, None)

    pltpu.sync_copy(degp, deg_sh.at[pl.ds(s * NP, NP)])
    plsc.subcore_barrier()

    n0 = s * NT
    for p in range(16):
        pltpu.sync_copy(deg_sh.at[pl.ds(p * NP + n0, NT)],
                        pbuf.at[pl.ds(p * NT, NT)])

    def red(k, _):
        acc = pbuf[pl.ds(k * 16, 16)]
        for p in range(1, 16):
            acc = acc + pbuf[pl.ds(p * NT + k * 16, 16)]
        dinv = jnp.where(acc == 0.0, 0.0, 1.0 / acc)
        dbuf[pl.ds(k * 16, 16)] = dinv
        return _
    lax.fori_loop(0, NT // 16, red, None)

    pltpu.sync_copy(dbuf, dinv_sh.at[pl.ds(n0, NT)])
    plsc.subcore_barrier()

    pltpu.sync_copy(dinv_sh, degp)
    wid = c * 16 + s

    def nchunk(k, _):
        e0 = wid * (EP // 32) + k * 2048
        pltpu.sync_copy(row_hbm.at[pl.ds(e0, 2048)], rbuf)
        pltpu.sync_copy(ew_hbm.at[pl.ds(e0, 2048)], wbuf)

        def body(i, _):
            r16 = rbuf[pl.ds(i * 16, 16)]
            w16 = wbuf[pl.ds(i * 16, 16)]
            d16 = plsc.load_gather(degp, [r16])
            nbuf[pl.ds(i * 16, 16)] = w16 * d16
            return _
        lax.fori_loop(0, 128, body, None)
        pltpu.sync_copy(nbuf, norm_hbm.at[pl.ds(e0, 2048)])
        return _
    lax.fori_loop(0, (EP // 32) // 2048, nchunk, None)


_norm_kernel = functools.partial(
    pl.kernel,
    out_type=jax.ShapeDtypeStruct((EP,), jnp.float32),
    mesh=_MESH,
    compiler_params=_SC_PARAMS,
    scratch_types=[
        pltpu.VMEM((NP,), jnp.float32),
        pltpu.VMEM((2048,), jnp.int32),
        pltpu.VMEM((2048,), jnp.float32),
        pltpu.VMEM((NP,), jnp.float32),
        pltpu.VMEM((NP // 16,), jnp.float32),
        pltpu.VMEM((2048,), jnp.float32),
        pltpu.VMEM_SHARED((16 * NP,), jnp.float32),
        pltpu.VMEM_SHARED((NP,), jnp.float32),
        pltpu.SemaphoreType.DMA,
    ],
)(_norm_body)


TPT = 80
GRP = 4
PAIR = 2 * GRP
NPAIR = TPT // PAIR
RPT = 640
RLAST = N - 15 * RPT


def _agg_body(h_hbm, row_hbm, col_hbm, norm_hbm, out_hbm,
              rlin0, clin0, nbuf0, ridx0, cidx0,
              rlin1, clin1, nbuf1, ridx1, cidx1,
              gbufA, gbufB, acc, sem_l, sem_a, sem_b, sem_sa, sem_sb):
    c = lax.axis_index("c")
    s = lax.axis_index("s")
    r0 = s * RPT
    L = [(rlin0, clin0, nbuf0, ridx0, cidx0),
         (rlin1, clin1, nbuf1, ridx1, cidx1)]
    G = [gbufA, gbufB]
    SG = [sem_a, sem_b]
    SS = [sem_sa, sem_sb]

    def zrow(r, _):
        for q in range(8):
            gbufA[r, pl.ds(q * 16, 16)] = jnp.zeros((16,), jnp.float32)
        return _
    lax.fori_loop(0, 128, zrow, None)

    @pl.when(s < 15)
    def _():
        for t in range(RPT // 128):
            pltpu.sync_copy(gbufA, acc.at[pl.ds(r0 + t * 128, 128)])

    @pl.when(s == 15)
    def _():
        for t in range((RLAST + 127) // 128):
            nr = min(128, RLAST - t * 128)
            pltpu.sync_copy(gbufA.at[pl.ds(0, nr)],
                            acc.at[pl.ds(15 * RPT + t * 128, nr)])
    plsc.subcore_barrier()

    ebase = c * (EP // 2) + s * (TPT * 128)

    def _load_group(g, gb):
        rlin, clin, nbuf, ridx, cidx = L[gb]
        e0 = ebase + g * (GRP * 128)
        cps = [pltpu.async_copy(row_hbm.at[pl.ds(e0, GRP * 128)], rlin, sem_l),
               pltpu.async_copy(col_hbm.at[pl.ds(e0, GRP * 128)], clin, sem_l),
               pltpu.async_copy(norm_hbm.at[pl.ds(e0, GRP * 128)], nbuf, sem_l)]
        for cp in cps:
            cp.wait()
        for i in range(GRP * 8):
            ridx[i // 8, pl.ds((i % 8) * 16, 16)] = rlin[pl.ds(i * 16, 16)]
            cidx[i // 8, pl.ds((i % 8) * 16, 16)] = clin[pl.ds(i * 16, 16)]

    def _gather(gb, t, b):
        ridx = L[gb][3]
        pltpu.async_copy(h_hbm.at[ridx.at[t]], G[b], SG[b])

    def _wait_scatter(pu):
        gb, t, b = (pu // GRP) % 2, pu % GRP, pu % 2
        pltpu.make_async_copy(G[b], acc.at[L[gb][4].at[t]], SS[b]).wait()

    def _process(gb, t, b):
        nbuf, cidx = L[gb][2], L[gb][4]
        gbuf = G[b]
        pltpu.make_async_copy(h_hbm.at[L[gb][3].at[t]], gbuf, SG[b]).wait()

        def sgrp(i, _):
            n16 = nbuf[pl.ds(t * 128 + i * 16, 16)]
            for u in range(16):
                j = i * 16 + u
                nb = n16[u]
                for q in range(8):
                    gbuf[j, pl.ds(q * 16, 16)] = (
                        gbuf[j, pl.ds(q * 16, 16)] * nb)
            return _
        lax.fori_loop(0, 8, sgrp, None)
        pltpu.async_copy(gbuf, acc.at[cidx.at[t]], SS[b], add=True)

    def pair(i, _):
        for gb in range(2):
            g = 2 * i + gb
            _load_group(g, gb)
            for t in range(GRP):
                u = gb * GRP + t
                b = u % 2
                if u < 2:
                    @pl.when(i > 0)
                    def _():
                        _wait_scatter(u - 2)
                else:
                    _wait_scatter(u - 2)
                _gather(gb, t, b)
                pu = u - 1
                pgb, pt, pb = (pu // GRP) % 2, pu % GRP, pu % 2
                if u == 0:
                    @pl.when(i > 0)
                    def _():
                        _process(pgb, pt, pb)
                else:
                    _process(pgb, pt, pb)
        return _
    lax.fori_loop(0, NPAIR, pair, None)
    _process(1, GRP - 1, (PAIR - 1) % 2)
    _wait_scatter(PAIR - 2)
    _wait_scatter(PAIR - 1)
    plsc.subcore_barrier()

    @pl.when(s < 15)
    def _():
        pltpu.sync_copy(acc.at[pl.ds(r0, RPT)],
                        out_hbm.at[pl.ds(c * N + r0, RPT)])

    @pl.when(s == 15)
    def _():
        pltpu.sync_copy(acc.at[pl.ds(15 * RPT, RLAST)],
                        out_hbm.at[pl.ds(c * N + 15 * RPT, RLAST)])


_agg_kernel = functools.partial(
    pl.kernel,
    out_type=jax.ShapeDtypeStruct((2 * N, D), jnp.float32),
    mesh=_MESH,
    compiler_params=_SC_PARAMS,
    scratch_types=[
        pltpu.VMEM((GRP * 128,), jnp.int32),
        pltpu.VMEM((GRP * 128,), jnp.int32),
        pltpu.VMEM((GRP * 128,), jnp.float32),
        pltpu.VMEM((GRP, 128), jnp.int32),
        pltpu.VMEM((GRP, 128), jnp.int32),
        pltpu.VMEM((GRP * 128,), jnp.int32),
        pltpu.VMEM((GRP * 128,), jnp.int32),
        pltpu.VMEM((GRP * 128,), jnp.float32),
        pltpu.VMEM((GRP, 128), jnp.int32),
        pltpu.VMEM((GRP, 128), jnp.int32),
        pltpu.VMEM((128, D), jnp.float32),
        pltpu.VMEM((128, D), jnp.float32),
        pltpu.VMEM_SHARED((N, D), jnp.float32),
        pltpu.SemaphoreType.DMA,
        pltpu.SemaphoreType.DMA,
        pltpu.SemaphoreType.DMA,
        pltpu.SemaphoreType.DMA,
        pltpu.SemaphoreType.DMA,
    ],
)(_agg_body)


def kernel(x, edge_index, edge_weight, add_self_loops, W1, b1, W2, b2):
    del add_self_loops
    pad = EP - E
    spread = (jnp.arange(pad, dtype=jnp.int32) * 16) % N
    row_p = jnp.concatenate([edge_index[0], spread])
    col_p = jnp.concatenate([edge_index[1], spread])
    ew_p = jnp.pad(edge_weight, (0, pad))

    norm = _norm_kernel(row_p, ew_p)
    h1 = _mm1(x, W1)
    p1 = _agg_kernel(h1, row_p, col_p, norm)
    h2 = _mm2(p1, b1, W2)
    p2 = _agg_kernel(h2, row_p, col_p, norm)
    return _fin(p2, b2)
